# Initial kernel scaffold; baseline (speedup 1.0000x reference)
import jax
import jax.numpy as jnp
from jax.experimental import pallas as pl


def kernel(x, table, W, b):
    raise NotImplementedError("write your pallas kernel here")


# speedup vs baseline: 13.2137x; 13.2137x over previous
import functools

import jax
import jax.numpy as jnp
from jax import lax
from jax.experimental import pallas as pl
from jax.experimental.pallas import tpu as pltpu
from jax.experimental.pallas import tpu_sc as plsc

BATCH = 4096
SEQ = 200
HIDDEN = 128
NCLASS = 100

NC = 2
NS = 16
NW = NC * NS
B_PER_W = BATCH // NW

CH0 = 128
CH1 = SEQ - CH0

_mesh = plsc.VectorSubcoreMesh(core_axis_name="c", subcore_axis_name="s")


@functools.partial(
    pl.kernel,
    mesh=_mesh,
    out_type=jax.ShapeDtypeStruct((BATCH, HIDDEN), jnp.float32),
    scratch_types=[
        pltpu.VMEM((B_PER_W, SEQ), jnp.int32),
        pltpu.VMEM((2, SEQ, HIDDEN), jnp.float32),
        pltpu.VMEM((B_PER_W, HIDDEN), jnp.float32),
        pltpu.SemaphoreType.DMA,
        pltpu.SemaphoreType.DMA,
    ],
)
def _pool_sc(x_hbm, table_hbm, out_hbm, idx_v, rows_v, sums_v, sem0, sem1):
    wid = lax.axis_index("s") * NC + lax.axis_index("c")
    base = wid * B_PER_W

    pltpu.sync_copy(x_hbm.at[pl.ds(base, B_PER_W)], idx_v)

    sems = (sem0, sem1)

    def start_row(r, buf):
        pltpu.async_copy(
            table_hbm.at[idx_v.at[r, pl.ds(0, CH0)]],
            rows_v.at[buf, pl.ds(0, CH0)],
            sems[buf],
        )
        pltpu.async_copy(
            table_hbm.at[idx_v.at[r, pl.ds(CH0, CH1)]],
            rows_v.at[buf, pl.ds(CH0, CH1)],
            sems[buf],
        )

    def wait_row(buf):
        pltpu.make_async_copy(
            table_hbm.at[pl.ds(0, SEQ)], rows_v.at[buf], sems[buf]
        ).wait()

    inv = jnp.float32(1.0 / SEQ)

    def accum_row(r, buf):
        def body(s, accs):
            return tuple(
                accs[h] + rows_v[buf, s, pl.ds(h * 16, 16)] for h in range(8)
            )
        accs = lax.fori_loop(
            0, SEQ, body, tuple(jnp.zeros((16,), jnp.float32) for _ in range(8))
        )
        for h in range(8):
            sums_v[r, pl.ds(h * 16, 16)] = accs[h] * inv

    start_row(0, 0)
    start_row(1, 1)

    def outer(ii, carry):
        r0 = 2 * ii
        wait_row(0)
        accum_row(r0, 0)

        @pl.when(ii < B_PER_W // 2 - 1)
        def _():
            start_row(r0 + 2, 0)

        wait_row(1)
        accum_row(r0 + 1, 1)

        @pl.when(ii < B_PER_W // 2 - 1)
        def _():
            start_row(r0 + 3, 1)

        return carry

    lax.fori_loop(0, B_PER_W // 2, outer, 0)

    pltpu.sync_copy(sums_v, out_hbm.at[pl.ds(base, B_PER_W)])


def _mm_body(p_ref, w_ref, b_ref, o_ref):
    o_ref[...] = (
        jnp.dot(p_ref[...], w_ref[...], preferred_element_type=jnp.float32)
        + b_ref[...]
    )


_mm = pl.pallas_call(
    _mm_body,
    grid=(8,),
    in_specs=[
        pl.BlockSpec((BATCH // 8, HIDDEN), lambda i: (i, 0)),
        pl.BlockSpec((HIDDEN, HIDDEN), lambda i: (0, 0)),
        pl.BlockSpec((1, HIDDEN), lambda i: (0, 0)),
    ],
    out_specs=pl.BlockSpec((BATCH // 8, HIDDEN), lambda i: (i, 0)),
    out_shape=jax.ShapeDtypeStruct((BATCH, HIDDEN), jnp.float32),
)


def kernel(x, table, W, b):
    pooled = _pool_sc(x, table)
    wt = jnp.pad(W, ((0, HIDDEN - NCLASS), (0, 0))).T
    bp = jnp.pad(b, (0, HIDDEN - NCLASS)).reshape(1, HIDDEN)
    out = _mm(pooled, wt, bp)
    return out[:, :NCLASS]
